# baseline (speedup 1.0000x reference)
import functools

import jax
import jax.numpy as jnp
from jax import lax
from jax.experimental import pallas as pl
from jax.experimental.pallas import tpu as pltpu
from jax.experimental.pallas import tpu_sc as plsc

B, H, W, C = 4, 224, 224, 96
NPIX = B * H * W
NW = 32
PPW = NPIX // NW
K = 64
NCHUNK = PPW // K
NBUF = 2
L = 16

_mesh = plsc.VectorSubcoreMesh(core_axis_name="c", subcore_axis_name="s")


@functools.partial(
    pl.kernel,
    mesh=_mesh,
    out_type=jax.ShapeDtypeStruct((NPIX, C), jnp.float32),
    scratch_types=(
        [pltpu.VMEM((PPW,), jnp.float32)] * 2
        + [pltpu.VMEM((4 * K,), jnp.int32)] * NBUF
        + [pltpu.VMEM((4 * K,), jnp.float32)] * NBUF
        + [pltpu.VMEM((4 * K, C), jnp.float32)] * NBUF
        + [pltpu.VMEM((K, C), jnp.float32)] * NBUF
        + [pltpu.SemaphoreType.DMA] * (2 * NBUF)
    ),
    compiler_params=pltpu.CompilerParams(
        needs_layout_passes=False, use_tc_tiling_on_sc=False),
)
def _resample_sc(table_hbm, coords_hbm, out_hbm, ys_v, xs_v, *scratch):
    idx_s = scratch[0:NBUF]
    w_s = scratch[NBUF:2 * NBUF]
    rows_s = scratch[2 * NBUF:3 * NBUF]
    out_s = scratch[3 * NBUF:4 * NBUF]
    gsem_s = scratch[4 * NBUF:5 * NBUF]
    osem_s = scratch[5 * NBUF:6 * NBUF]

    wid = lax.axis_index("s") * 2 + lax.axis_index("c")
    pbase = wid * PPW
    boff = (pbase // (H * W)) * (H * W)
    pltpu.sync_copy(coords_hbm.at[0, pl.ds(pbase, PPW)], ys_v)
    pltpu.sync_copy(coords_hbm.at[1, pl.ds(pbase, PPW)], xs_v)
    lane = lax.iota(jnp.int32, L)

    def prep(j, b):
        for h in range(K // L):
            y = ys_v[pl.ds(j * K + h * L, L)]
            x = xs_v[pl.ds(j * K + h * L, L)]
            y0 = jnp.clip(y.astype(jnp.int32), 0, H - 2)
            x0 = jnp.clip(x.astype(jnp.int32), 0, W - 2)
            wy = y - y0.astype(jnp.float32)
            wx = x - x0.astype(jnp.float32)
            base = boff + y0 * W + x0
            idx_s[b][pl.ds(0 * K + h * L, L)] = base
            idx_s[b][pl.ds(1 * K + h * L, L)] = base + 1
            idx_s[b][pl.ds(2 * K + h * L, L)] = base + W
            idx_s[b][pl.ds(3 * K + h * L, L)] = base + W + 1
            w_s[b][pl.ds(0 * K + h * L, L)] = (1.0 - wy) * (1.0 - wx)
            w_s[b][pl.ds(1 * K + h * L, L)] = (1.0 - wy) * wx
            w_s[b][pl.ds(2 * K + h * L, L)] = wy * (1.0 - wx)
            w_s[b][pl.ds(3 * K + h * L, L)] = wy * wx
        pltpu.make_async_copy(
            table_hbm.at[idx_s[b]], rows_s[b], gsem_s[b]).start()

    def blend(b):
        for h in range(K // L):
            w00 = w_s[b][pl.ds(0 * K + h * L, L)]
            w01 = w_s[b][pl.ds(1 * K + h * L, L)]
            w10 = w_s[b][pl.ds(2 * K + h * L, L)]
            w11 = w_s[b][pl.ds(3 * K + h * L, L)]
            prow = h * L + lane
            r0 = prow
            r1 = prow + K
            r2 = prow + 2 * K
            r3 = prow + 3 * K

            def cbody(c, _, w00=w00, w01=w01, w10=w10, w11=w11,
                      r0=r0, r1=r1, r2=r2, r3=r3, prow=prow):
                col = jnp.full((L,), c, jnp.int32)
                a = plsc.load_gather(rows_s[b], [r0, col])
                bb = plsc.load_gather(rows_s[b], [r1, col])
                cc = plsc.load_gather(rows_s[b], [r2, col])
                d = plsc.load_gather(rows_s[b], [r3, col])
                o = w00 * a + w01 * bb + w10 * cc + w11 * d
                plsc.store_scatter(out_s[b], [prow, col], o)
                return _

            lax.fori_loop(0, C, cbody, 0, unroll=8)

    for b in range(NBUF - 1):
        prep(b, b)

    def chunk_group(g, carry):
        for b in range(NBUF):
            j = g * NBUF + b
            pltpu.make_async_copy(
                table_hbm.at[idx_s[b]], rows_s[b], gsem_s[b]).wait()

            @pl.when(j >= NBUF)
            def _wait_out(b=b, j=j):
                pltpu.make_async_copy(
                    out_s[b], out_hbm.at[pl.ds(pbase + (j - NBUF) * K, K)],
                    osem_s[b]).wait()

            blend(b)
            pltpu.make_async_copy(
                out_s[b], out_hbm.at[pl.ds(pbase + j * K, K)],
                osem_s[b]).start()

            @pl.when(j + NBUF - 1 < NCHUNK)
            def _prep_next(b=b, j=j):
                prep(j + NBUF - 1, (b + NBUF - 1) % NBUF)
        return carry

    lax.fori_loop(0, NCHUNK // NBUF, chunk_group, 0)

    for b in range(NBUF):
        pltpu.make_async_copy(
            out_s[b],
            out_hbm.at[pl.ds(pbase + (NCHUNK - NBUF + b) * K, K)],
            osem_s[b]).wait()


def kernel(inputs, sample_coords):
    table = inputs.reshape(B * H * W, C)
    coords = jnp.moveaxis(sample_coords.reshape(NPIX, 2), -1, 0)
    out = _resample_sc(table, coords)
    return out.reshape(B, H, W, C)


# speedup vs baseline: 1.1485x; 1.1485x over previous
import functools

import jax
import jax.numpy as jnp
from jax import lax
from jax.experimental import pallas as pl
from jax.experimental.pallas import tpu as pltpu
from jax.experimental.pallas import tpu_sc as plsc

B, H, W, C = 4, 224, 224, 96
CP = 128
NPIX = B * H * W
NW = 32
PPW = NPIX // NW
K = 32
NCHUNK = PPW // K
NBUF = 4
L = 16

_mesh = plsc.VectorSubcoreMesh(core_axis_name="c", subcore_axis_name="s")


@functools.partial(
    pl.kernel,
    mesh=_mesh,
    out_type=jax.ShapeDtypeStruct((NPIX, CP), jnp.float32),
    scratch_types=(
        [pltpu.VMEM((PPW,), jnp.float32)] * 2
        + [pltpu.VMEM((4 * K,), jnp.int32)] * NBUF
        + [pltpu.VMEM((4 * K,), jnp.float32)] * NBUF
        + [pltpu.VMEM((4 * K, CP), jnp.float32)] * NBUF
        + [pltpu.VMEM((K, CP), jnp.float32)] * NBUF
        + [pltpu.SemaphoreType.DMA] * (2 * NBUF)
    ),
    compiler_params=pltpu.CompilerParams(needs_layout_passes=False),
)
def _resample_sc(table_hbm, coords_hbm, out_hbm, ys_v, xs_v, *scratch):
    idx_s = scratch[0:NBUF]
    w_s = scratch[NBUF:2 * NBUF]
    rows_s = scratch[2 * NBUF:3 * NBUF]
    out_s = scratch[3 * NBUF:4 * NBUF]
    gsem_s = scratch[4 * NBUF:5 * NBUF]
    osem_s = scratch[5 * NBUF:6 * NBUF]

    wid = lax.axis_index("s") * 2 + lax.axis_index("c")
    pbase = wid * PPW
    boff = (pbase // (H * W)) * (H * W)
    pltpu.sync_copy(coords_hbm.at[0, pl.ds(pbase, PPW)], ys_v)
    pltpu.sync_copy(coords_hbm.at[1, pl.ds(pbase, PPW)], xs_v)
    lane = lax.iota(jnp.int32, L)

    def prep(j, b):
        for h in range(K // L):
            y = ys_v[pl.ds(j * K + h * L, L)]
            x = xs_v[pl.ds(j * K + h * L, L)]
            y0 = jnp.clip(y.astype(jnp.int32), 0, H - 2)
            x0 = jnp.clip(x.astype(jnp.int32), 0, W - 2)
            wy = y - y0.astype(jnp.float32)
            wx = x - x0.astype(jnp.float32)
            base = boff + y0 * W + x0
            for k, off in enumerate((0, 1, W, W + 1)):
                pltpu.make_async_copy(
                    table_hbm.at[base + off],
                    rows_s[b].at[pl.ds(k * K + h * L, L)],
                    gsem_s[b]).start()
            w_s[b][pl.ds(0 * K + h * L, L)] = (1.0 - wy) * (1.0 - wx)
            w_s[b][pl.ds(1 * K + h * L, L)] = (1.0 - wy) * wx
            w_s[b][pl.ds(2 * K + h * L, L)] = wy * (1.0 - wx)
            w_s[b][pl.ds(3 * K + h * L, L)] = wy * wx

    def blend(b):
        for h in range(K // L):
            w00 = w_s[b][pl.ds(0 * K + h * L, L)]
            w01 = w_s[b][pl.ds(1 * K + h * L, L)]
            w10 = w_s[b][pl.ds(2 * K + h * L, L)]
            w11 = w_s[b][pl.ds(3 * K + h * L, L)]
            prow = h * L + lane
            r0 = prow
            r1 = prow + K
            r2 = prow + 2 * K
            r3 = prow + 3 * K

            def cbody(c, _, w00=w00, w01=w01, w10=w10, w11=w11,
                      r0=r0, r1=r1, r2=r2, r3=r3, prow=prow):
                col = jnp.full((L,), c, jnp.int32)
                a = plsc.load_gather(rows_s[b], [r0, col])
                bb = plsc.load_gather(rows_s[b], [r1, col])
                cc = plsc.load_gather(rows_s[b], [r2, col])
                d = plsc.load_gather(rows_s[b], [r3, col])
                o = w00 * a + w01 * bb + w10 * cc + w11 * d
                plsc.store_scatter(out_s[b], [prow, col], o)
                return _

            lax.fori_loop(0, C, cbody, 0, unroll=8)

    for b in range(NBUF - 1):
        prep(b, b)

    def chunk_group(g, carry):
        for b in range(NBUF):
            j = g * NBUF + b
            pltpu.make_async_copy(
                table_hbm.at[idx_s[b]], rows_s[b], gsem_s[b]).wait()

            @pl.when(j >= NBUF)
            def _wait_out(b=b, j=j):
                pltpu.make_async_copy(
                    out_s[b], out_hbm.at[pl.ds(pbase + (j - NBUF) * K, K)],
                    osem_s[b]).wait()

            blend(b)
            pltpu.make_async_copy(
                out_s[b], out_hbm.at[pl.ds(pbase + j * K, K)],
                osem_s[b]).start()

            @pl.when(j + NBUF - 1 < NCHUNK)
            def _prep_next(b=b, j=j):
                prep(j + NBUF - 1, (b + NBUF - 1) % NBUF)
        return carry

    lax.fori_loop(0, NCHUNK // NBUF, chunk_group, 0)

    for b in range(NBUF):
        pltpu.make_async_copy(
            out_s[b],
            out_hbm.at[pl.ds(pbase + (NCHUNK - NBUF + b) * K, K)],
            osem_s[b]).wait()


def kernel(inputs, sample_coords):
    table = jnp.pad(inputs.reshape(NPIX, C), ((0, 0), (0, CP - C)))
    coords = jnp.moveaxis(sample_coords.reshape(NPIX, 2), -1, 0)
    out = _resample_sc(table, coords)
    return out[:, :C].reshape(B, H, W, C)


# speedup vs baseline: 1.3718x; 1.1944x over previous
import functools

import jax
import jax.numpy as jnp
from jax import lax
from jax.experimental import pallas as pl
from jax.experimental.pallas import tpu as pltpu
from jax.experimental.pallas import tpu_sc as plsc

B, H, W, C = 4, 224, 224, 96
CW = C // 2
NPIX = B * H * W
NW = 32
PPW = NPIX // NW
K = 32
NCHUNK = PPW // K
NBUF = 4
L = 16

_mesh = plsc.VectorSubcoreMesh(core_axis_name="c", subcore_axis_name="s")


@functools.partial(
    pl.kernel,
    mesh=_mesh,
    out_type=jax.ShapeDtypeStruct((NPIX, CW), jnp.int32),
    scratch_types=(
        [pltpu.VMEM((PPW,), jnp.float32)] * 2
        + [pltpu.VMEM((4 * K,), jnp.int32)] * NBUF
        + [pltpu.VMEM((4 * K,), jnp.float32)] * NBUF
        + [pltpu.VMEM((4 * K, CW), jnp.int32)] * NBUF
        + [pltpu.VMEM((K, CW), jnp.int32)] * NBUF
        + [pltpu.SemaphoreType.DMA] * (2 * NBUF)
    ),
    compiler_params=pltpu.CompilerParams(
        needs_layout_passes=False, use_tc_tiling_on_sc=False),
)
def _resample_sc(table_hbm, coords_hbm, out_hbm, ys_v, xs_v, *scratch):
    idx_s = scratch[0:NBUF]
    w_s = scratch[NBUF:2 * NBUF]
    rows_s = scratch[2 * NBUF:3 * NBUF]
    out_s = scratch[3 * NBUF:4 * NBUF]
    gsem_s = scratch[4 * NBUF:5 * NBUF]
    osem_s = scratch[5 * NBUF:6 * NBUF]

    wid = lax.axis_index("s") * 2 + lax.axis_index("c")
    pbase = wid * PPW
    boff = (pbase // (H * W)) * (H * W)
    pltpu.sync_copy(coords_hbm.at[0, pl.ds(pbase, PPW)], ys_v)
    pltpu.sync_copy(coords_hbm.at[1, pl.ds(pbase, PPW)], xs_v)
    lane = lax.iota(jnp.int32, L)

    def prep(j, b):
        for h in range(K // L):
            y = ys_v[pl.ds(j * K + h * L, L)]
            x = xs_v[pl.ds(j * K + h * L, L)]
            y0 = jnp.clip(y.astype(jnp.int32), 0, H - 2)
            x0 = jnp.clip(x.astype(jnp.int32), 0, W - 2)
            wy = y - y0.astype(jnp.float32)
            wx = x - x0.astype(jnp.float32)
            base = boff + y0 * W + x0
            for k, off in enumerate((0, 1, W, W + 1)):
                pltpu.make_async_copy(
                    table_hbm.at[base + off],
                    rows_s[b].at[pl.ds(k * K + h * L, L)],
                    gsem_s[b]).start()
            w_s[b][pl.ds(0 * K + h * L, L)] = (1.0 - wy) * (1.0 - wx)
            w_s[b][pl.ds(1 * K + h * L, L)] = (1.0 - wy) * wx
            w_s[b][pl.ds(2 * K + h * L, L)] = wy * (1.0 - wx)
            w_s[b][pl.ds(3 * K + h * L, L)] = wy * wx

    def blend(b):
        for h in range(K // L):
            w00 = w_s[b][pl.ds(0 * K + h * L, L)]
            w01 = w_s[b][pl.ds(1 * K + h * L, L)]
            w10 = w_s[b][pl.ds(2 * K + h * L, L)]
            w11 = w_s[b][pl.ds(3 * K + h * L, L)]
            prow = h * L + lane
            rr = [prow, prow + K, prow + 2 * K, prow + 3 * K]

            def cbody(c, _, w00=w00, w01=w01, w10=w10, w11=w11,
                      rr=rr, prow=prow):
                col = jnp.full((L,), c, jnp.int32)
                ev, od = [], []
                for r in rr:
                    v = plsc.load_gather(rows_s[b], [r, col])
                    e, o = plsc.unpack(
                        plsc.bitcast(v, jnp.bfloat16),
                        format=plsc.PackFormat.INTERLEAVED)
                    ev.append(e)
                    od.append(o)
                oe = w00 * ev[0] + w01 * ev[1] + w10 * ev[2] + w11 * ev[3]
                oo = w00 * od[0] + w01 * od[1] + w10 * od[2] + w11 * od[3]
                packed = plsc.bitcast(
                    plsc.pack(oe, oo, format=plsc.PackFormat.INTERLEAVED),
                    jnp.int32)
                plsc.store_scatter(out_s[b], [prow, col], packed)
                return _

            lax.fori_loop(0, CW, cbody, 0, unroll=8)

    for b in range(NBUF - 1):
        prep(b, b)

    def chunk_group(g, carry):
        for b in range(NBUF):
            j = g * NBUF + b
            pltpu.make_async_copy(
                table_hbm.at[idx_s[b]], rows_s[b], gsem_s[b]).wait()

            @pl.when(j >= NBUF)
            def _wait_out(b=b, j=j):
                pltpu.make_async_copy(
                    out_s[b], out_hbm.at[pl.ds(pbase + (j - NBUF) * K, K)],
                    osem_s[b]).wait()

            blend(b)
            pltpu.make_async_copy(
                out_s[b], out_hbm.at[pl.ds(pbase + j * K, K)],
                osem_s[b]).start()

            @pl.when(j + NBUF - 1 < NCHUNK)
            def _prep_next(b=b, j=j):
                prep(j + NBUF - 1, (b + NBUF - 1) % NBUF)
        return carry

    lax.fori_loop(0, NCHUNK // NBUF, chunk_group, 0)

    for b in range(NBUF):
        pltpu.make_async_copy(
            out_s[b],
            out_hbm.at[pl.ds(pbase + (NCHUNK - NBUF + b) * K, K)],
            osem_s[b]).wait()


def kernel(inputs, sample_coords):
    table = lax.bitcast_convert_type(
        inputs.astype(jnp.bfloat16).reshape(NPIX, CW, 2), jnp.int32)
    coords = jnp.moveaxis(sample_coords.reshape(NPIX, 2), -1, 0)
    out = _resample_sc(table, coords)
    out = lax.bitcast_convert_type(out, jnp.bfloat16).astype(jnp.float32)
    return out.reshape(B, H, W, C)


# speedup vs baseline: 1.4602x; 1.0644x over previous
import functools

import jax
import jax.numpy as jnp
from jax import lax
from jax.experimental import pallas as pl
from jax.experimental.pallas import tpu as pltpu
from jax.experimental.pallas import tpu_sc as plsc

B, H, W, C = 4, 224, 224, 96
CW = C // 2
NPIX = B * H * W
NW = 32
PPW = NPIX // NW
K = 32
NCHUNK = PPW // K
NBUF = 4
L = 16

_mesh = plsc.VectorSubcoreMesh(core_axis_name="c", subcore_axis_name="s")


@functools.partial(
    pl.kernel,
    mesh=_mesh,
    out_type=jax.ShapeDtypeStruct((NPIX, C), jnp.float32),
    scratch_types=(
        [pltpu.VMEM((PPW,), jnp.float32)] * 2
        + [pltpu.VMEM((4 * K,), jnp.int32)] * NBUF
        + [pltpu.VMEM((4 * K,), jnp.float32)] * NBUF
        + [pltpu.VMEM((4 * K, CW), jnp.int32)] * NBUF
        + [pltpu.VMEM((K, C), jnp.float32)] * NBUF
        + [pltpu.SemaphoreType.DMA] * (2 * NBUF)
    ),
    compiler_params=pltpu.CompilerParams(
        needs_layout_passes=False, use_tc_tiling_on_sc=False),
)
def _resample_sc(table_hbm, coords_hbm, out_hbm, ys_v, xs_v, *scratch):
    idx_s = scratch[0:NBUF]
    w_s = scratch[NBUF:2 * NBUF]
    rows_s = scratch[2 * NBUF:3 * NBUF]
    out_s = scratch[3 * NBUF:4 * NBUF]
    gsem_s = scratch[4 * NBUF:5 * NBUF]
    osem_s = scratch[5 * NBUF:6 * NBUF]

    wid = lax.axis_index("s") * 2 + lax.axis_index("c")
    pbase = wid * PPW
    boff = (pbase // (H * W)) * (H * W)
    pltpu.sync_copy(coords_hbm.at[0, pl.ds(pbase, PPW)], ys_v)
    pltpu.sync_copy(coords_hbm.at[1, pl.ds(pbase, PPW)], xs_v)
    lane = lax.iota(jnp.int32, L)

    def prep(j, b):
        for h in range(K // L):
            y = ys_v[pl.ds(j * K + h * L, L)]
            x = xs_v[pl.ds(j * K + h * L, L)]
            y0 = jnp.clip(y.astype(jnp.int32), 0, H - 2)
            x0 = jnp.clip(x.astype(jnp.int32), 0, W - 2)
            wy = y - y0.astype(jnp.float32)
            wx = x - x0.astype(jnp.float32)
            base = boff + y0 * W + x0
            for k, off in enumerate((0, 1, W, W + 1)):
                pltpu.make_async_copy(
                    table_hbm.at[base + off],
                    rows_s[b].at[pl.ds(k * K + h * L, L)],
                    gsem_s[b]).start()
            w_s[b][pl.ds(0 * K + h * L, L)] = (1.0 - wy) * (1.0 - wx)
            w_s[b][pl.ds(1 * K + h * L, L)] = (1.0 - wy) * wx
            w_s[b][pl.ds(2 * K + h * L, L)] = wy * (1.0 - wx)
            w_s[b][pl.ds(3 * K + h * L, L)] = wy * wx

    def blend(b):
        for h in range(K // L):
            w00 = w_s[b][pl.ds(0 * K + h * L, L)]
            w01 = w_s[b][pl.ds(1 * K + h * L, L)]
            w10 = w_s[b][pl.ds(2 * K + h * L, L)]
            w11 = w_s[b][pl.ds(3 * K + h * L, L)]
            prow = h * L + lane
            rr = [prow, prow + K, prow + 2 * K, prow + 3 * K]

            def cbody(c, _, w00=w00, w01=w01, w10=w10, w11=w11,
                      rr=rr, prow=prow):
                col = jnp.full((L,), c, jnp.int32)
                ev, od = [], []
                for r in rr:
                    v = plsc.load_gather(rows_s[b], [r, col])
                    e, o = plsc.unpack(
                        plsc.bitcast(v, jnp.bfloat16),
                        format=plsc.PackFormat.INTERLEAVED)
                    ev.append(e)
                    od.append(o)
                oe = w00 * ev[0] + w01 * ev[1] + w10 * ev[2] + w11 * ev[3]
                oo = w00 * od[0] + w01 * od[1] + w10 * od[2] + w11 * od[3]
                col2 = col + col
                plsc.store_scatter(out_s[b], [prow, col2], oe)
                plsc.store_scatter(out_s[b], [prow, col2 + 1], oo)
                return _

            lax.fori_loop(0, CW, cbody, 0, unroll=8)

    for b in range(NBUF - 1):
        prep(b, b)

    def chunk_group(g, carry):
        for b in range(NBUF):
            j = g * NBUF + b
            pltpu.make_async_copy(
                table_hbm.at[idx_s[b]], rows_s[b], gsem_s[b]).wait()

            @pl.when(j >= NBUF)
            def _wait_out(b=b, j=j):
                pltpu.make_async_copy(
                    out_s[b], out_hbm.at[pl.ds(pbase + (j - NBUF) * K, K)],
                    osem_s[b]).wait()

            blend(b)
            pltpu.make_async_copy(
                out_s[b], out_hbm.at[pl.ds(pbase + j * K, K)],
                osem_s[b]).start()

            @pl.when(j + NBUF - 1 < NCHUNK)
            def _prep_next(b=b, j=j):
                prep(j + NBUF - 1, (b + NBUF - 1) % NBUF)
        return carry

    lax.fori_loop(0, NCHUNK // NBUF, chunk_group, 0)

    for b in range(NBUF):
        pltpu.make_async_copy(
            out_s[b],
            out_hbm.at[pl.ds(pbase + (NCHUNK - NBUF + b) * K, K)],
            osem_s[b]).wait()


def kernel(inputs, sample_coords):
    table = lax.bitcast_convert_type(
        inputs.astype(jnp.bfloat16).reshape(NPIX, CW, 2), jnp.int32)
    coords = jnp.moveaxis(sample_coords.reshape(NPIX, 2), -1, 0)
    out = _resample_sc(table, coords)
    return out.reshape(B, H, W, C)


# speedup vs baseline: 1.8420x; 1.2615x over previous
import functools

import jax
import jax.numpy as jnp
from jax import lax
from jax.experimental import pallas as pl
from jax.experimental.pallas import tpu as pltpu
from jax.experimental.pallas import tpu_sc as plsc

B, H, W, C = 4, 224, 224, 96
CW = C // 2
NPIX = B * H * W
NW = 32
PPW = NPIX // NW
K = 32
NCHUNK = PPW // K
NBUF = 4
L = 16

_mesh = plsc.VectorSubcoreMesh(core_axis_name="c", subcore_axis_name="s")


@functools.partial(
    pl.kernel,
    mesh=_mesh,
    out_type=jax.ShapeDtypeStruct((NPIX, CW), jnp.int32),
    scratch_types=(
        [pltpu.VMEM((PPW,), jnp.float32)] * 2
        + [pltpu.VMEM((4 * K,), jnp.int32)] * NBUF
        + [pltpu.VMEM((4 * K,), jnp.float32)] * NBUF
        + [pltpu.VMEM((4 * K, CW), jnp.int32)] * NBUF
        + [pltpu.VMEM((K, CW), jnp.int32)] * NBUF
        + [pltpu.SemaphoreType.DMA] * (2 * NBUF)
    ),
    compiler_params=pltpu.CompilerParams(
        needs_layout_passes=False, use_tc_tiling_on_sc=False),
)
def _resample_sc(table_hbm, coords_hbm, out_hbm, ys_v, xs_v, *scratch):
    idx_s = scratch[0:NBUF]
    w_s = scratch[NBUF:2 * NBUF]
    rows_s = scratch[2 * NBUF:3 * NBUF]
    out_s = scratch[3 * NBUF:4 * NBUF]
    gsem_s = scratch[4 * NBUF:5 * NBUF]
    osem_s = scratch[5 * NBUF:6 * NBUF]

    wid = lax.axis_index("s") * 2 + lax.axis_index("c")
    pbase = wid * PPW
    boff = (pbase // (H * W)) * (H * W)
    pltpu.sync_copy(coords_hbm.at[0, pl.ds(pbase, PPW)], ys_v)
    pltpu.sync_copy(coords_hbm.at[1, pl.ds(pbase, PPW)], xs_v)
    lane = lax.iota(jnp.int32, L)

    def prep(j, b):
        for h in range(K // L):
            y = ys_v[pl.ds(j * K + h * L, L)]
            x = xs_v[pl.ds(j * K + h * L, L)]
            y0 = jnp.clip(y.astype(jnp.int32), 0, H - 2)
            x0 = jnp.clip(x.astype(jnp.int32), 0, W - 2)
            wy = y - y0.astype(jnp.float32)
            wx = x - x0.astype(jnp.float32)
            base = boff + y0 * W + x0
            for k, off in enumerate((0, 1, W, W + 1)):
                pltpu.make_async_copy(
                    table_hbm.at[base + off],
                    rows_s[b].at[pl.ds(k * K + h * L, L)],
                    gsem_s[b]).start()
            w_s[b][pl.ds(0 * K + h * L, L)] = (1.0 - wy) * (1.0 - wx)
            w_s[b][pl.ds(1 * K + h * L, L)] = (1.0 - wy) * wx
            w_s[b][pl.ds(2 * K + h * L, L)] = wy * (1.0 - wx)
            w_s[b][pl.ds(3 * K + h * L, L)] = wy * wx

    def blend(b):
        for h in range(K // L):
            w00 = w_s[b][pl.ds(0 * K + h * L, L)]
            w01 = w_s[b][pl.ds(1 * K + h * L, L)]
            w10 = w_s[b][pl.ds(2 * K + h * L, L)]
            w11 = w_s[b][pl.ds(3 * K + h * L, L)]
            prow = h * L + lane
            rr = [prow, prow + K, prow + 2 * K, prow + 3 * K]

            def cbody(c, _, w00=w00, w01=w01, w10=w10, w11=w11,
                      rr=rr, prow=prow):
                col = jnp.full((L,), c, jnp.int32)
                ev, od = [], []
                for r in rr:
                    v = plsc.load_gather(rows_s[b], [r, col])
                    e, o = plsc.unpack(
                        plsc.bitcast(v, jnp.bfloat16),
                        format=plsc.PackFormat.INTERLEAVED)
                    ev.append(e)
                    od.append(o)
                oe = w00 * ev[0] + w01 * ev[1] + w10 * ev[2] + w11 * ev[3]
                oo = w00 * od[0] + w01 * od[1] + w10 * od[2] + w11 * od[3]
                packed = plsc.bitcast(
                    plsc.pack(oe, oo, format=plsc.PackFormat.INTERLEAVED),
                    jnp.int32)
                plsc.store_scatter(out_s[b], [prow, col], packed)
                return _

            lax.fori_loop(0, CW, cbody, 0, unroll=8)

    for b in range(NBUF - 1):
        prep(b, b)

    def chunk_group(g, carry):
        for b in range(NBUF):
            j = g * NBUF + b
            pltpu.make_async_copy(
                table_hbm.at[idx_s[b]], rows_s[b], gsem_s[b]).wait()

            @pl.when(j >= NBUF)
            def _wait_out(b=b, j=j):
                pltpu.make_async_copy(
                    out_s[b], out_hbm.at[pl.ds(pbase + (j - NBUF) * K, K)],
                    osem_s[b]).wait()

            blend(b)
            pltpu.make_async_copy(
                out_s[b], out_hbm.at[pl.ds(pbase + j * K, K)],
                osem_s[b]).start()

            @pl.when(j + NBUF - 1 < NCHUNK)
            def _prep_next(b=b, j=j):
                prep(j + NBUF - 1, (b + NBUF - 1) % NBUF)
        return carry

    lax.fori_loop(0, NCHUNK // NBUF, chunk_group, 0)

    for b in range(NBUF):
        pltpu.make_async_copy(
            out_s[b],
            out_hbm.at[pl.ds(pbase + (NCHUNK - NBUF + b) * K, K)],
            osem_s[b]).wait()


def kernel(inputs, sample_coords):
    v = lax.bitcast_convert_type(inputs.reshape(NPIX, C), jnp.uint32)
    v = v + jnp.uint32(0x8000)
    table = lax.bitcast_convert_type(
        (v[:, :CW] >> 16) | (v[:, CW:] & jnp.uint32(0xFFFF0000)), jnp.int32)
    coords = jnp.moveaxis(sample_coords.reshape(NPIX, 2), -1, 0)
    out = _resample_sc(table, coords)
    w = lax.bitcast_convert_type(out, jnp.uint32)
    lo = lax.bitcast_convert_type(w << 16, jnp.float32)
    hi = lax.bitcast_convert_type(w & jnp.uint32(0xFFFF0000), jnp.float32)
    return jnp.concatenate([lo, hi], axis=1).reshape(B, H, W, C)


# speedup vs baseline: 2.1920x; 1.1900x over previous
import functools

import jax
import jax.numpy as jnp
from jax import lax
from jax.experimental import pallas as pl
from jax.experimental.pallas import tpu as pltpu
from jax.experimental.pallas import tpu_sc as plsc

B, H, W, C = 4, 224, 224, 96
CW = C // 2
NPIX = B * H * W
NW = 32
PPW = NPIX // NW
K = 32
NCHUNK = PPW // K
NBUF = 4
L = 16

_mesh = plsc.VectorSubcoreMesh(core_axis_name="c", subcore_axis_name="s")


@functools.partial(
    pl.kernel,
    mesh=_mesh,
    out_type=jax.ShapeDtypeStruct((NPIX, CW), jnp.int32),
    scratch_types=(
        [pltpu.VMEM((2 * PPW,), jnp.float32)]
        + [pltpu.VMEM((4 * K,), jnp.int32)] * NBUF
        + [pltpu.VMEM((4 * K,), jnp.float32)] * NBUF
        + [pltpu.VMEM((4 * K, CW), jnp.int32)] * NBUF
        + [pltpu.VMEM((K, CW), jnp.int32)] * NBUF
        + [pltpu.SemaphoreType.DMA] * (2 * NBUF)
    ),
    compiler_params=pltpu.CompilerParams(
        needs_layout_passes=False, use_tc_tiling_on_sc=False),
)
def _resample_sc(table_hbm, coords_hbm, out_hbm, cv_v, *scratch):
    idx_s = scratch[0:NBUF]
    w_s = scratch[NBUF:2 * NBUF]
    rows_s = scratch[2 * NBUF:3 * NBUF]
    out_s = scratch[3 * NBUF:4 * NBUF]
    gsem_s = scratch[4 * NBUF:5 * NBUF]
    osem_s = scratch[5 * NBUF:6 * NBUF]

    wid = lax.axis_index("s") * 2 + lax.axis_index("c")
    pbase = wid * PPW
    boff = (pbase // (H * W)) * (H * W)
    pltpu.sync_copy(coords_hbm.at[pl.ds(2 * pbase, 2 * PPW)], cv_v)
    lane = lax.iota(jnp.int32, L)

    def prep(j, b):
        for h in range(K // L):
            pv2 = 2 * (j * K + h * L) + 2 * lane
            y = plsc.load_gather(cv_v, [pv2])
            x = plsc.load_gather(cv_v, [pv2 + 1])
            y0 = jnp.clip(y.astype(jnp.int32), 0, H - 2)
            x0 = jnp.clip(x.astype(jnp.int32), 0, W - 2)
            wy = y - y0.astype(jnp.float32)
            wx = x - x0.astype(jnp.float32)
            base = boff + y0 * W + x0
            for k, off in enumerate((0, 1, W, W + 1)):
                pltpu.make_async_copy(
                    table_hbm.at[base + off],
                    rows_s[b].at[pl.ds(k * K + h * L, L)],
                    gsem_s[b]).start()
            w_s[b][pl.ds(0 * K + h * L, L)] = (1.0 - wy) * (1.0 - wx)
            w_s[b][pl.ds(1 * K + h * L, L)] = (1.0 - wy) * wx
            w_s[b][pl.ds(2 * K + h * L, L)] = wy * (1.0 - wx)
            w_s[b][pl.ds(3 * K + h * L, L)] = wy * wx

    def blend(b):
        for h in range(K // L):
            w00 = w_s[b][pl.ds(0 * K + h * L, L)]
            w01 = w_s[b][pl.ds(1 * K + h * L, L)]
            w10 = w_s[b][pl.ds(2 * K + h * L, L)]
            w11 = w_s[b][pl.ds(3 * K + h * L, L)]
            prow = h * L + lane
            rr = [prow, prow + K, prow + 2 * K, prow + 3 * K]

            def cbody(c, _, w00=w00, w01=w01, w10=w10, w11=w11,
                      rr=rr, prow=prow):
                col = jnp.full((L,), c, jnp.int32)
                ev, od = [], []
                for r in rr:
                    v = plsc.load_gather(rows_s[b], [r, col])
                    e, o = plsc.unpack(
                        plsc.bitcast(v, jnp.bfloat16),
                        format=plsc.PackFormat.INTERLEAVED)
                    ev.append(e)
                    od.append(o)
                oe = w00 * ev[0] + w01 * ev[1] + w10 * ev[2] + w11 * ev[3]
                oo = w00 * od[0] + w01 * od[1] + w10 * od[2] + w11 * od[3]
                packed = plsc.bitcast(
                    plsc.pack(oe, oo, format=plsc.PackFormat.INTERLEAVED),
                    jnp.int32)
                plsc.store_scatter(out_s[b], [prow, col], packed)
                return _

            lax.fori_loop(0, CW, cbody, 0, unroll=8)

    for b in range(NBUF - 1):
        prep(b, b)

    def chunk_group(g, carry):
        for b in range(NBUF):
            j = g * NBUF + b
            pltpu.make_async_copy(
                table_hbm.at[idx_s[b]], rows_s[b], gsem_s[b]).wait()

            @pl.when(j >= NBUF)
            def _wait_out(b=b, j=j):
                pltpu.make_async_copy(
                    out_s[b], out_hbm.at[pl.ds(pbase + (j - NBUF) * K, K)],
                    osem_s[b]).wait()

            blend(b)
            pltpu.make_async_copy(
                out_s[b], out_hbm.at[pl.ds(pbase + j * K, K)],
                osem_s[b]).start()

            @pl.when(j + NBUF - 1 < NCHUNK)
            def _prep_next(b=b, j=j):
                prep(j + NBUF - 1, (b + NBUF - 1) % NBUF)
        return carry

    lax.fori_loop(0, NCHUNK // NBUF, chunk_group, 0)

    for b in range(NBUF):
        pltpu.make_async_copy(
            out_s[b],
            out_hbm.at[pl.ds(pbase + (NCHUNK - NBUF + b) * K, K)],
            osem_s[b]).wait()


def kernel(inputs, sample_coords):
    v = lax.bitcast_convert_type(inputs, jnp.uint32) + jnp.uint32(0x8000)
    table = lax.bitcast_convert_type(
        (v[..., :CW] >> 16) | (v[..., CW:] & jnp.uint32(0xFFFF0000)),
        jnp.int32).reshape(NPIX, CW)
    coords = sample_coords.reshape(2 * NPIX)
    out = _resample_sc(table, coords)
    w = lax.bitcast_convert_type(out, jnp.uint32)
    lo = lax.bitcast_convert_type(w << 16, jnp.float32)
    hi = lax.bitcast_convert_type(w & jnp.uint32(0xFFFF0000), jnp.float32)
    return jnp.concatenate([lo, hi], axis=1).reshape(B, H, W, C)
